# baseline (speedup 1.0000x reference)
import functools
import math

import jax
import jax.numpy as jnp
from jax import lax
from jax.experimental import pallas as pl
from jax.experimental.pallas import tpu as pltpu
from jax.experimental.pallas import tpu_sc as plsc

_B = 256
_S = 1000
_V = 655
_D = 1024
_NCH = 6
_NC, _NS, _L = 2, 16, 16
_NW = _NC * _NS
_RW = _B // _NW
_FULL = _S // _L
_TAIL = _S - _FULL * _L
_CNT = _NCH * _RW * 128
_ZC = _CNT // _L
_BATCH = 8
_JIN = _FULL // _BATCH
_REM = _FULL - _JIN * _BATCH


def _hist_body(tok_hbm, out_hbm, tok_v, cnt_v, sem, osem):
    wid = lax.axis_index("s") * _NC + lax.axis_index("c")
    cp = pltpu.make_async_copy(
        tok_hbm.at[pl.ds(_RW * wid, _RW)], tok_v, sem)
    cp.start()
    zeros = jnp.zeros((_L,), jnp.float32)
    ones = jnp.ones((_L,), jnp.float32)

    def _zero(k, _):
        cnt_v[pl.ds(k * _L, _L)] = zeros
        return None

    lax.fori_loop(0, _ZC, _zero, None, unroll=4)
    cp.wait()

    tail_mask = lax.iota(jnp.int32, _L) >= _L - _TAIL

    def _row(r, _):
        rbase = jnp.broadcast_to(r * 128, (_L,)).astype(jnp.int32)

        def _scat(j, _, r=r, rbase=rbase):
            ts = [tok_v[r, pl.ds(j * _BATCH * _L + s * _L, _L)]
                  for s in range(_BATCH)]
            idxs = [t + (t >> 7) * 896 + rbase for t in ts]
            for idx in idxs:
                plsc.addupdate_scatter(cnt_v, [idx], ones)
            return None

        lax.fori_loop(0, _JIN, _scat, None)
        ts = [tok_v[r, pl.ds((_JIN * _BATCH + s) * _L, _L)]
              for s in range(_REM)]
        ts.append(tok_v[r, pl.ds(_S - _L, _L)])
        idxs = [t + (t >> 7) * 896 + rbase for t in ts]
        for s in range(_REM):
            plsc.addupdate_scatter(cnt_v, [idxs[s]], ones)
        plsc.addupdate_scatter(cnt_v, [idxs[_REM]], ones, mask=tail_mask)
        return None

    lax.fori_loop(0, _RW, _row, None)

    ocps = [
        pltpu.make_async_copy(
            cnt_v.at[pl.ds(ch * _RW * 128, _RW * 128)],
            out_hbm.at[pl.ds((ch * _B + _RW * wid) * 128, _RW * 128)],
            osem)
        for ch in range(_NCH)
    ]
    for cp2 in ocps:
        cp2.start()
    for cp2 in ocps:
        cp2.wait()


@functools.partial(
    pl.kernel,
    mesh=plsc.VectorSubcoreMesh(core_axis_name="c", subcore_axis_name="s"),
    out_type=jax.ShapeDtypeStruct((_NCH * _B * 128,), jnp.float32),
    scratch_types=[
        pltpu.VMEM((_RW, _S), jnp.int32),
        pltpu.VMEM((_CNT,), jnp.float32),
        pltpu.SemaphoreType.DMA,
        pltpu.SemaphoreType.DMA,
    ],
    compiler_params=pltpu.CompilerParams(needs_layout_passes=False),
)
def _histogram(tok_hbm, out_hbm, tok_v, cnt_v, sem, osem):
    _hist_body(tok_hbm, out_hbm, tok_v, cnt_v, sem, osem)


_INV_SQRT2 = 1.0 / math.sqrt(2.0)


def _gelu(x):
    return x * 0.5 * (1.0 + lax.erf(x * _INV_SQRT2))


def _prep_body(tbl_ref, wp_ref, out_ref):
    out_ref[...] = lax.dot(tbl_ref[...], wp_ref[...],
                           preferred_element_type=jnp.float32)


def _dense_body(cnt_ref, t2_ref, bp_ref, g_ref, be_ref,
                wm_ref, bm_ref, wf_ref, bf_ref, out_ref):
    pooled = lax.dot(cnt_ref[5 * _B:6 * _B, :_V - 5 * 128],
                     t2_ref[5 * 128:_V, :], preferred_element_type=jnp.float32)
    for c in range(5):
        pooled += lax.dot(cnt_ref[c * _B:(c + 1) * _B, :],
                          t2_ref[c * 128:(c + 1) * 128, :],
                          preferred_element_type=jnp.float32)
    h = pooled * (1.0 / _S) + bp_ref[...][None, :]
    mu = jnp.mean(h, axis=-1, keepdims=True)
    d = h - mu
    var = jnp.mean(d * d, axis=-1, keepdims=True)
    x = d * lax.rsqrt(var + 1e-5) * g_ref[...][None, :] + be_ref[...][None, :]
    x = lax.dot(x, wm_ref[...], preferred_element_type=jnp.float32) + bm_ref[...][None, :]
    x = _gelu(x)
    x = lax.dot(x, wf_ref[...], preferred_element_type=jnp.float32) + bf_ref[...][None, :]
    out_ref[...] = _gelu(x)


def kernel(tokens, table, W_pool, b_pool, gamma, beta, W_mlp, b_mlp, W_fc1, b_fc1):
    tokens = tokens.astype(jnp.int32)
    t2 = pl.pallas_call(
        _prep_body,
        out_shape=jax.ShapeDtypeStruct((_V, _D), jnp.float32),
    )(table, W_pool)
    counts = _histogram(tokens).reshape(_NCH * _B, 128)
    out = pl.pallas_call(
        _dense_body,
        out_shape=jax.ShapeDtypeStruct((_B, 768), jnp.float32),
    )(counts, t2, b_pool, gamma, beta, W_mlp, b_mlp, W_fc1, b_fc1)
    return out


# speedup vs baseline: 1.0191x; 1.0191x over previous
import functools
import math

import jax
import jax.numpy as jnp
from jax import lax
from jax.experimental import pallas as pl
from jax.experimental.pallas import tpu as pltpu
from jax.experimental.pallas import tpu_sc as plsc

_B = 256
_S = 1000
_V = 655
_D = 1024
_NCH = 6
_NC, _NS, _L = 1, 16, 16
_NW = _NC * _NS
_RW = _B // _NW
_FULL = _S // _L
_TAIL = _S - _FULL * _L
_CNT = _NCH * _RW * 128
_ZC = _CNT // _L
_BATCH = 8
_JIN = _FULL // _BATCH
_REM = _FULL - _JIN * _BATCH


def _hist_body(tok_hbm, out_hbm, tok_v, cnt_v, sem, osem):
    wid = lax.axis_index("s") * _NC + lax.axis_index("c")
    cp = pltpu.make_async_copy(
        tok_hbm.at[pl.ds(_RW * wid, _RW)], tok_v, sem)
    cp.start()
    zeros = jnp.zeros((_L,), jnp.float32)
    ones = jnp.ones((_L,), jnp.float32)

    def _zero(k, _):
        cnt_v[pl.ds(k * _L, _L)] = zeros
        return None

    lax.fori_loop(0, _ZC, _zero, None, unroll=4)
    cp.wait()

    tail_mask = lax.iota(jnp.int32, _L) >= _L - _TAIL

    def _row(r, _):
        rbase = jnp.broadcast_to(r * 128, (_L,)).astype(jnp.int32)

        def _scat(j, _, r=r, rbase=rbase):
            ts = [tok_v[r, pl.ds(j * _BATCH * _L + s * _L, _L)]
                  for s in range(_BATCH)]
            idxs = [t + (t >> 7) * 896 + rbase for t in ts]
            for idx in idxs:
                plsc.addupdate_scatter(cnt_v, [idx], ones)
            return None

        lax.fori_loop(0, _JIN, _scat, None)
        ts = [tok_v[r, pl.ds((_JIN * _BATCH + s) * _L, _L)]
              for s in range(_REM)]
        ts.append(tok_v[r, pl.ds(_S - _L, _L)])
        idxs = [t + (t >> 7) * 896 + rbase for t in ts]
        for s in range(_REM):
            plsc.addupdate_scatter(cnt_v, [idxs[s]], ones)
        plsc.addupdate_scatter(cnt_v, [idxs[_REM]], ones, mask=tail_mask)
        return None

    lax.fori_loop(0, _RW, _row, None)

    ocps = [
        pltpu.make_async_copy(
            cnt_v.at[pl.ds(ch * _RW * 128, _RW * 128)],
            out_hbm.at[pl.ds((ch * _B + _RW * wid) * 128, _RW * 128)],
            osem)
        for ch in range(_NCH)
    ]
    for cp2 in ocps:
        cp2.start()
    for cp2 in ocps:
        cp2.wait()


@functools.partial(
    pl.kernel,
    mesh=plsc.VectorSubcoreMesh(core_axis_name="c", subcore_axis_name="s", num_cores=1),
    out_type=jax.ShapeDtypeStruct((_NCH * _B * 128,), jnp.float32),
    scratch_types=[
        pltpu.VMEM((_RW, _S), jnp.int32),
        pltpu.VMEM((_CNT,), jnp.float32),
        pltpu.SemaphoreType.DMA,
        pltpu.SemaphoreType.DMA,
    ],
    compiler_params=pltpu.CompilerParams(needs_layout_passes=False),
)
def _histogram(tok_hbm, out_hbm, tok_v, cnt_v, sem, osem):
    _hist_body(tok_hbm, out_hbm, tok_v, cnt_v, sem, osem)


_INV_SQRT2 = 1.0 / math.sqrt(2.0)


def _gelu(x):
    return x * 0.5 * (1.0 + lax.erf(x * _INV_SQRT2))


def _prep_body(tbl_ref, wp_ref, out_ref):
    out_ref[...] = lax.dot(tbl_ref[...], wp_ref[...],
                           preferred_element_type=jnp.float32)


def _dense_body(cnt_ref, t2_ref, bp_ref, g_ref, be_ref,
                wm_ref, bm_ref, wf_ref, bf_ref, out_ref):
    pooled = lax.dot(cnt_ref[5 * _B:6 * _B, :_V - 5 * 128],
                     t2_ref[5 * 128:_V, :], preferred_element_type=jnp.float32)
    for c in range(5):
        pooled += lax.dot(cnt_ref[c * _B:(c + 1) * _B, :],
                          t2_ref[c * 128:(c + 1) * 128, :],
                          preferred_element_type=jnp.float32)
    h = pooled * (1.0 / _S) + bp_ref[...][None, :]
    mu = jnp.mean(h, axis=-1, keepdims=True)
    d = h - mu
    var = jnp.mean(d * d, axis=-1, keepdims=True)
    x = d * lax.rsqrt(var + 1e-5) * g_ref[...][None, :] + be_ref[...][None, :]
    x = lax.dot(x, wm_ref[...], preferred_element_type=jnp.float32) + bm_ref[...][None, :]
    x = _gelu(x)
    x = lax.dot(x, wf_ref[...], preferred_element_type=jnp.float32) + bf_ref[...][None, :]
    out_ref[...] = _gelu(x)


def kernel(tokens, table, W_pool, b_pool, gamma, beta, W_mlp, b_mlp, W_fc1, b_fc1):
    tokens = tokens.astype(jnp.int32)
    t2 = pl.pallas_call(
        _prep_body,
        out_shape=jax.ShapeDtypeStruct((_V, _D), jnp.float32),
    )(table, W_pool)
    counts = _histogram(tokens).reshape(_NCH * _B, 128)
    out = pl.pallas_call(
        _dense_body,
        out_shape=jax.ShapeDtypeStruct((_B, 768), jnp.float32),
    )(counts, t2, b_pool, gamma, beta, W_mlp, b_mlp, W_fc1, b_fc1)
    return out
